# baseline (speedup 1.0000x reference)
import jax
import jax.numpy as jnp
from jax import lax
from jax.experimental import pallas as pl
from jax.experimental.pallas import tpu as pltpu
from jax.experimental.pallas import tpu_sc as plsc

N = 100000
F = 128
G = 512
NC = 2
NS = 16
NW = NC * NS
CHUNK = 80
NCHUNKS = N // CHUNK
NGRP = CHUNK // 16
BASE_CH = NCHUNKS // NW
EXTRA = NCHUNKS - BASE_CH * NW
MAXCH = BASE_CH + 1
NK = F // 16


def _hsum(s, iota16):
    dn = lax.GatherDimensionNumbers(
        offset_dims=(), collapsed_slice_dims=(0,), start_index_map=(0,))
    for sh in (8, 4, 2, 1):
        s = s + lax.gather(s, (iota16 ^ sh)[:, None], dn, (1,),
                           mode=lax.GatherScatterMode.PROMISE_IN_BOUNDS)
    return s[0]


def _sc_kernel(x_hbm, i_hbm, a_hbm, out_hbm, md_hbm,
               xbuf, ibuf, av, acc, lg):
    cid = lax.axis_index("c")
    sid = lax.axis_index("s")
    wg = cid * NS + sid
    nch = BASE_CH + (wg < EXTRA).astype(jnp.int32)
    iota16 = lax.iota(jnp.int32, 16)

    pltpu.sync_copy(a_hbm, av)
    a_regs = [av[pl.ds(16 * k, 16)] for k in range(NK)]

    def _z(g, carry):
        for k in range(NK):
            acc[g, pl.ds(16 * k, 16)] = jnp.zeros((16,), jnp.float32)
        return carry
    lax.fori_loop(0, G, _z, 0)

    def _chunkA(j, m_run):
        row0 = (wg + NW * j) * CHUNK
        pltpu.sync_copy(x_hbm.at[pl.ds(row0, CHUNK)], xbuf)
        m_acc = m_run
        for g in range(NGRP):
            lvec = jnp.zeros((16,), jnp.float32)
            for i in range(16):
                r = 16 * g + i
                s = jnp.zeros((16,), jnp.float32)
                for k in range(NK):
                    s = s + xbuf[r, pl.ds(16 * k, 16)] * a_regs[k]
                l = _hsum(s, iota16)
                m_acc = jnp.maximum(m_acc, l)
                lvec = lvec + jnp.where(iota16 == i,
                                        jnp.full((16,), l, jnp.float32), 0.0)
            lg[pl.ds(j * CHUNK + 16 * g, 16)] = lvec
        return m_acc

    m_w = lax.fori_loop(0, nch, _chunkA, jnp.float32(-3.0e38))

    def _chunkB(j, d_run):
        row0 = (wg + NW * j) * CHUNK
        pltpu.sync_copy(x_hbm.at[pl.ds(row0, CHUNK)], xbuf)
        pltpu.sync_copy(i_hbm.at[pl.ds(row0, CHUNK)], ibuf)
        d_acc = d_run
        for g in range(NGRP):
            lvec = lg[pl.ds(j * CHUNK + 16 * g, 16)]
            wvec = jnp.exp(lvec - m_w)
            gvec = ibuf[pl.ds(16 * g, 16)]
            d_acc = d_acc + wvec
            for i in range(16):
                r = 16 * g + i
                gid = gvec[i]
                ws = jnp.full((16,), wvec[i], jnp.float32)
                for k in range(NK):
                    sl = pl.ds(16 * k, 16)
                    plsc.addupdate(acc.at[gid, sl], ws * xbuf[r, sl])
        return d_acc

    d_vec = lax.fori_loop(0, nch, _chunkB, jnp.zeros((16,), jnp.float32))
    d_w = _hsum(d_vec, iota16)

    md = (jnp.where(iota16 == 0, jnp.full((16,), m_w, jnp.float32), 0.0)
          + jnp.where(iota16 == 1, jnp.full((16,), d_w, jnp.float32), 0.0))
    xbuf[0, pl.ds(0, 16)] = md
    pltpu.sync_copy(xbuf.at[0, pl.ds(0, 16)], md_hbm.at[wg])
    pltpu.sync_copy(acc, out_hbm.at[wg])


def kernel(X, I, attn_kernel):
    mesh = plsc.VectorSubcoreMesh(core_axis_name="c", subcore_axis_name="s")
    k = pl.kernel(
        _sc_kernel,
        mesh=mesh,
        out_type=[
            jax.ShapeDtypeStruct((NW, G, F), jnp.float32),
            jax.ShapeDtypeStruct((NW, 16), jnp.float32),
        ],
        scratch_types=[
            pltpu.VMEM((CHUNK, F), jnp.float32),
            pltpu.VMEM((CHUNK,), jnp.int32),
            pltpu.VMEM((F,), jnp.float32),
            pltpu.VMEM((G, F), jnp.float32),
            pltpu.VMEM((MAXCH * CHUNK,), jnp.float32),
        ],
    )
    parts, md = k(X, I.astype(jnp.int32), attn_kernel.reshape(F))
    m = md[:, 0]
    d = md[:, 1]
    mg = jnp.max(m)
    s = jnp.exp(m - mg)
    num = jnp.einsum("w,wgf->gf", s, parts)
    den = jnp.dot(d, s)
    return num / den


# speedup vs baseline: 10.4889x; 10.4889x over previous
import jax
import jax.numpy as jnp
from jax.experimental import pallas as pl
from jax.experimental.pallas import tpu as pltpu

N = 100000
F = 128
G = 512
TILE = 20000
GRID = N // TILE
WIN = 128


def _body(meta_ref, x_ref, i_ref, a_ref, o_ref, acc_ref, m_ref, d_ref):
    step = pl.program_id(0)

    @pl.when(step == 0)
    def _init():
        m_ref[0, 0] = -jnp.inf
        d_ref[0, 0] = 0.0
        acc_ref[...] = jnp.zeros_like(acc_ref)

    x = x_ref[...]
    logits = jax.lax.dot_general(
        a_ref[...], x, (((0,), (1,)), ((), ())),
        preferred_element_type=jnp.float32)
    m_old = m_ref[0, 0]
    m_new = jnp.maximum(m_old, jnp.max(logits))
    m_ref[0, 0] = m_new
    scale = jnp.exp(m_old - m_new)
    w = jnp.exp(logits - m_new)
    d_ref[0, 0] = d_ref[0, 0] * scale + jnp.sum(w)

    @pl.when(m_new > m_old)
    def _rescale():
        acc_ref[...] = acc_ref[...] * scale

    iv = i_ref[0]
    wb = w.astype(jnp.bfloat16)
    t = x.astype(jnp.bfloat16)
    goff = meta_ref[step, 0]
    ok = meta_ref[step, 1] != 0

    @pl.when(ok)
    def _windowed():
        rows = jax.lax.broadcasted_iota(jnp.int16, (WIN, 1), 0)
        local = (iv - goff).astype(jnp.int16)
        pt = jnp.where(rows == local, wb, jnp.bfloat16(0))
        contrib = jax.lax.dot_general(
            pt, t, (((1,), (0,)), ((), ())), preferred_element_type=jnp.float32)
        acc_ref[pl.ds(goff, WIN), :] = acc_ref[pl.ds(goff, WIN), :] + contrib

    @pl.when(jnp.logical_not(ok))
    def _full():
        rows = jax.lax.broadcasted_iota(jnp.int16, (G, 1), 0)
        pt = jnp.where(rows == iv.astype(jnp.int16),
                       wb, jnp.bfloat16(0))
        contrib = jax.lax.dot_general(
            pt, t, (((1,), (0,)), ((), ())), preferred_element_type=jnp.float32)
        acc_ref[...] = acc_ref[...] + contrib

    @pl.when(step == GRID - 1)
    def _finish():
        o_ref[...] = acc_ref[...] / d_ref[0, 0]


def kernel(X, I, attn_kernel):
    Ii = I.astype(jnp.int32)
    starts = Ii[::TILE]
    ends = Ii[TILE - 1::TILE]
    goff = jnp.minimum((starts // 8) * 8, G - WIN)
    ok = (ends - goff) < WIN
    meta = jnp.stack([goff, ok.astype(jnp.int32)], axis=1)
    I3 = Ii.reshape(GRID, 1, TILE)
    return pl.pallas_call(
        _body,
        grid=(GRID,),
        in_specs=[
            pl.BlockSpec(memory_space=pltpu.SMEM),
            pl.BlockSpec((TILE, F), lambda i: (i, 0)),
            pl.BlockSpec((1, 1, TILE), lambda i: (i, 0, 0)),
            pl.BlockSpec((F, 1), lambda i: (0, 0)),
        ],
        out_specs=pl.BlockSpec((G, F), lambda i: (0, 0)),
        out_shape=jax.ShapeDtypeStruct((G, F), jnp.float32),
        scratch_shapes=[
            pltpu.VMEM((G, F), jnp.float32),
            pltpu.SMEM((1, 1), jnp.float32),
            pltpu.SMEM((1, 1), jnp.float32),
        ],
        compiler_params=pltpu.CompilerParams(
            dimension_semantics=("arbitrary",),
        ),
    )(meta, X, I3, attn_kernel)
